# baseline (speedup 1.0000x reference)
import functools

import jax
import jax.numpy as jnp
from jax import lax
from jax.experimental import pallas as pl
from jax.experimental.pallas import tpu as pltpu
from jax.experimental.pallas import tpu_sc as plsc

K_BUCKETS = 1000
EMBED_DIM = 128
BATCH = 16384

_NC = 2
_NS = 16
_NW = _NC * _NS
_B_PER_W = BATCH // _NW

_NB = 4
_NCHUNK = 8
_CHUNK = _B_PER_W // _NCHUNK

_mesh = plsc.VectorSubcoreMesh(core_axis_name="c", subcore_axis_name="s")


@functools.partial(
    pl.kernel,
    mesh=_mesh,
    out_type=jax.ShapeDtypeStruct((BATCH, EMBED_DIM), jnp.float32),
    scratch_types=[
        pltpu.VMEM((_B_PER_W,), jnp.int32),
        pltpu.VMEM((_CHUNK, EMBED_DIM), jnp.float32),
        pltpu.VMEM((_CHUNK, EMBED_DIM), jnp.float32),
        pltpu.VMEM((_CHUNK, EMBED_DIM), jnp.float32),
        pltpu.VMEM((_CHUNK, EMBED_DIM), jnp.float32),
        pltpu.SemaphoreType.DMA,
        pltpu.SemaphoreType.DMA,
    ],
)
def _gather_kernel(ids_hbm, table_hbm, out_hbm, idx_v, b0, b1, b2, b3, gsem, ssem):
    wid = lax.axis_index("s") * _NC + lax.axis_index("c")
    base = wid * _B_PER_W
    bufs = (b0, b1, b2, b3)

    def start_gather(j):
        return pltpu.async_copy(
            table_hbm.at[idx_v.at[pl.ds(j * _CHUNK, _CHUNK)]], bufs[j % _NB], gsem
        )

    def start_store(j):
        return pltpu.async_copy(
            bufs[j % _NB], out_hbm.at[pl.ds(base + j * _CHUNK, _CHUNK)], ssem
        )

    pltpu.sync_copy(ids_hbm.at[pl.ds(base, _B_PER_W)], idx_v)
    gathers = {0: start_gather(0), 1: start_gather(1)}
    stores = {}
    for k in range(_NCHUNK):
        gathers[k].wait()
        stores[k] = start_store(k)
        if k + 2 < _NCHUNK:
            if k - 2 >= 0:
                stores[k - 2].wait()
            gathers[k + 2] = start_gather(k + 2)
    for k in range(max(0, _NCHUNK - 4), _NCHUNK):
        stores[k].wait()


def kernel(ids, embed_weight):
    return _gather_kernel(ids.astype(jnp.int32), embed_weight)


# speedup vs baseline: 1.1548x; 1.1548x over previous
import functools

import jax
import jax.numpy as jnp
from jax import lax
from jax.experimental import pallas as pl
from jax.experimental.pallas import tpu as pltpu
from jax.experimental.pallas import tpu_sc as plsc

K_BUCKETS = 1000
EMBED_DIM = 128
BATCH = 16384

_NC = 2
_NS = 16
_NW = _NC * _NS
_B_PER_W = BATCH // _NW

_NB = 2
_NCHUNK = 4
_CHUNK = _B_PER_W // _NCHUNK

_STAGE = 64

_mesh = plsc.VectorSubcoreMesh(core_axis_name="c", subcore_axis_name="s")


@functools.partial(
    pl.kernel,
    mesh=_mesh,
    out_type=jax.ShapeDtypeStruct((BATCH, EMBED_DIM), jnp.float32),
    scratch_types=[
        pltpu.VMEM((_B_PER_W,), jnp.int32),
        pltpu.VMEM((_CHUNK, EMBED_DIM), jnp.float32),
        pltpu.VMEM((_CHUNK, EMBED_DIM), jnp.float32),
        pltpu.VMEM((_STAGE, EMBED_DIM), jnp.float32),
        pltpu.VMEM_SHARED((K_BUCKETS, EMBED_DIM), jnp.float32),
        pltpu.SemaphoreType.DMA,
        pltpu.SemaphoreType.DMA,
    ],
)
def _gather_kernel(ids_hbm, table_hbm, out_hbm, idx_v, b0, b1, stage, tbl_sp,
                   gsem, ssem):
    cid = lax.axis_index("c")
    sid = lax.axis_index("s")
    wid = sid * _NC + cid
    base = wid * _B_PER_W
    bufs = (b0, b1)

    @pl.when(sid < _NS - 1)
    def _():
        row0 = sid * _STAGE
        pltpu.sync_copy(table_hbm.at[pl.ds(row0, _STAGE)], stage)
        pltpu.sync_copy(stage, tbl_sp.at[pl.ds(row0, _STAGE)])

    @pl.when(sid == _NS - 1)
    def _():
        last = (_NS - 1) * _STAGE
        pltpu.sync_copy(table_hbm.at[pl.ds(last, K_BUCKETS - last)],
                        stage.at[pl.ds(0, K_BUCKETS - last)])
        pltpu.sync_copy(stage.at[pl.ds(0, K_BUCKETS - last)],
                        tbl_sp.at[pl.ds(last, K_BUCKETS - last)])

    pltpu.sync_copy(ids_hbm.at[pl.ds(base, _B_PER_W)], idx_v)
    plsc.subcore_barrier()

    def start_gather(j):
        return pltpu.async_copy(
            tbl_sp.at[idx_v.at[pl.ds(j * _CHUNK, _CHUNK)]], bufs[j % _NB], gsem
        )

    def start_store(j):
        return pltpu.async_copy(
            bufs[j % _NB], out_hbm.at[pl.ds(base + j * _CHUNK, _CHUNK)], ssem
        )

    gathers = {0: start_gather(0)}
    stores = {}
    for k in range(_NCHUNK):
        gathers[k].wait()
        stores[k] = start_store(k)
        if k - 1 >= 0:
            stores[k - 1].wait()
        if k + 1 < _NCHUNK:
            gathers[k + 1] = start_gather(k + 1)
    stores[_NCHUNK - 1].wait()


def kernel(ids, embed_weight):
    return _gather_kernel(ids.astype(jnp.int32), embed_weight)


# speedup vs baseline: 1.1550x; 1.0002x over previous
import functools

import jax
import jax.numpy as jnp
from jax import lax
from jax.experimental import pallas as pl
from jax.experimental.pallas import tpu as pltpu
from jax.experimental.pallas import tpu_sc as plsc

K_BUCKETS = 1000
EMBED_DIM = 128
BATCH = 16384

_NC = 2
_NS = 16
_NW = _NC * _NS
_B_PER_W = BATCH // _NW

_NB = 2
_NCHUNK = 4
_CHUNK = _B_PER_W // _NCHUNK

_STAGE = 64

_mesh = plsc.VectorSubcoreMesh(core_axis_name="c", subcore_axis_name="s")


@functools.partial(
    pl.kernel,
    mesh=_mesh,
    out_type=jax.ShapeDtypeStruct((BATCH, EMBED_DIM), jnp.float32),
    scratch_types=[
        pltpu.VMEM((_B_PER_W,), jnp.int32),
        pltpu.VMEM((_CHUNK, EMBED_DIM), jnp.float32),
        pltpu.VMEM((_CHUNK, EMBED_DIM), jnp.float32),
        pltpu.VMEM_SHARED((K_BUCKETS, EMBED_DIM), jnp.float32),
        pltpu.SemaphoreType.DMA,
        pltpu.SemaphoreType.DMA,
    ],
)
def _gather_kernel(ids_hbm, table_hbm, out_hbm, idx_v, b0, b1, tbl_sp,
                   gsem, ssem):
    cid = lax.axis_index("c")
    sid = lax.axis_index("s")
    wid = sid * _NC + cid
    base = wid * _B_PER_W
    bufs = (b0, b1)

    @pl.when(sid < _NS - 1)
    def _():
        row0 = sid * _STAGE
        pltpu.sync_copy(table_hbm.at[pl.ds(row0, _STAGE)],
                        tbl_sp.at[pl.ds(row0, _STAGE)])

    @pl.when(sid == _NS - 1)
    def _():
        last = (_NS - 1) * _STAGE
        pltpu.sync_copy(table_hbm.at[pl.ds(last, K_BUCKETS - last)],
                        tbl_sp.at[pl.ds(last, K_BUCKETS - last)])

    pltpu.sync_copy(ids_hbm.at[pl.ds(base, _B_PER_W)], idx_v)
    plsc.subcore_barrier()

    def start_gather(j):
        return pltpu.async_copy(
            tbl_sp.at[idx_v.at[pl.ds(j * _CHUNK, _CHUNK)]], bufs[j % _NB], gsem
        )

    def start_store(j):
        return pltpu.async_copy(
            bufs[j % _NB], out_hbm.at[pl.ds(base + j * _CHUNK, _CHUNK)], ssem
        )

    gathers = {0: start_gather(0)}
    stores = {}
    for k in range(_NCHUNK):
        gathers[k].wait()
        stores[k] = start_store(k)
        if k - 1 >= 0:
            stores[k - 1].wait()
        if k + 1 < _NCHUNK:
            gathers[k + 1] = start_gather(k + 1)
    stores[_NCHUNK - 1].wait()


def kernel(ids, embed_weight):
    return _gather_kernel(ids.astype(jnp.int32), embed_weight)


# speedup vs baseline: 1.1611x; 1.0053x over previous
import functools

import jax
import jax.numpy as jnp
from jax import lax
from jax.experimental import pallas as pl
from jax.experimental.pallas import tpu as pltpu
from jax.experimental.pallas import tpu_sc as plsc

K_BUCKETS = 1000
EMBED_DIM = 128
BATCH = 16384

_NC = 2
_NS = 16
_NW = _NC * _NS
_B_PER_W = BATCH // _NW

_NB = 4
_NCHUNK = 8
_CHUNK = _B_PER_W // _NCHUNK

_STAGE = 64

_mesh = plsc.VectorSubcoreMesh(core_axis_name="c", subcore_axis_name="s")


@functools.partial(
    pl.kernel,
    mesh=_mesh,
    out_type=jax.ShapeDtypeStruct((BATCH, EMBED_DIM), jnp.float32),
    scratch_types=[
        pltpu.VMEM((_B_PER_W,), jnp.int32),
        pltpu.VMEM((_CHUNK, EMBED_DIM), jnp.float32),
        pltpu.VMEM((_CHUNK, EMBED_DIM), jnp.float32),
        pltpu.VMEM((_CHUNK, EMBED_DIM), jnp.float32),
        pltpu.VMEM((_CHUNK, EMBED_DIM), jnp.float32),
        pltpu.VMEM_SHARED((K_BUCKETS, EMBED_DIM), jnp.float32),
        pltpu.SemaphoreType.DMA,
        pltpu.SemaphoreType.DMA,
    ],
)
def _gather_kernel(ids_hbm, table_hbm, out_hbm, idx_v, b0, b1, b2, b3, tbl_sp,
                   gsem, ssem):
    cid = lax.axis_index("c")
    sid = lax.axis_index("s")
    wid = sid * _NC + cid
    base = wid * _B_PER_W
    bufs = (b0, b1, b2, b3)

    @pl.when(sid < _NS - 1)
    def _():
        row0 = sid * _STAGE
        pltpu.sync_copy(table_hbm.at[pl.ds(row0, _STAGE)],
                        tbl_sp.at[pl.ds(row0, _STAGE)])

    @pl.when(sid == _NS - 1)
    def _():
        last = (_NS - 1) * _STAGE
        pltpu.sync_copy(table_hbm.at[pl.ds(last, K_BUCKETS - last)],
                        tbl_sp.at[pl.ds(last, K_BUCKETS - last)])

    pltpu.sync_copy(ids_hbm.at[pl.ds(base, _B_PER_W)], idx_v)
    plsc.subcore_barrier()

    def start_gather(j):
        return pltpu.async_copy(
            tbl_sp.at[idx_v.at[pl.ds(j * _CHUNK, _CHUNK)]], bufs[j % _NB], gsem
        )

    def start_store(j):
        return pltpu.async_copy(
            bufs[j % _NB], out_hbm.at[pl.ds(base + j * _CHUNK, _CHUNK)], ssem
        )

    gathers = {0: start_gather(0)}
    stores = {}
    for k in range(_NCHUNK):
        gathers[k].wait()
        stores[k] = start_store(k)
        j = k + 1
        if j < _NCHUNK:
            if j - _NB >= 0:
                stores[j - _NB].wait()
            gathers[j] = start_gather(j)
    for k in range(_NCHUNK - _NB, _NCHUNK):
        stores[k].wait()


def kernel(ids, embed_weight):
    return _gather_kernel(ids.astype(jnp.int32), embed_weight)
